# baseline (device time: 83200 ns/iter reference)
import jax
import jax.numpy as jnp
from jax import lax
from jax.experimental import pallas as pl
from jax.experimental.pallas import tpu as pltpu

N_CHUNKS = 8


def kernel(x, pi):
    n_rows = x.shape[1]
    rows_per = n_rows // N_CHUNKS

    def body(
        pi_ref,
        x_ref,
        out_ref,
        q_send,
        q_recv,
        s_send,
        s_recv,
        q_send_sems,
        q_recv_sems,
        s_send_sems,
        s_recv_sems,
    ):
        my_pos = lax.axis_index("i")
        dst = my_pos

        def make_rdmas(c):
            rows = pl.ds(c * rows_per, rows_per)
            rdma_q = pltpu.make_async_remote_copy(
                src_ref=q_send.at[:, rows, :],
                dst_ref=q_recv.at[:, rows, :],
                send_sem=q_send_sems.at[c],
                recv_sem=q_recv_sems.at[c],
                device_id=dst,
                device_id_type=pl.DeviceIdType.LOGICAL,
            )
            rdma_s = pltpu.make_async_remote_copy(
                src_ref=s_send.at[:, rows],
                dst_ref=s_recv.at[:, rows],
                send_sem=s_send_sems.at[c],
                recv_sem=s_recv_sems.at[c],
                device_id=dst,
                device_id_type=pl.DeviceIdType.LOGICAL,
            )
            return rdma_q, rdma_s

        for c in range(N_CHUNKS):
            rows = pl.ds(c * rows_per, rows_per)
            xv = x_ref[:, rows, :]
            scale = jnp.max(jnp.abs(xv), axis=2) / 127.0
            s_send[:, rows] = scale
            inv = 1.0 / jnp.maximum(scale, 1e-30)
            q_send[:, rows, :] = jnp.rint(xv * inv[:, :, None]).astype(jnp.int8)
            rdma_q, rdma_s = make_rdmas(c)
            rdma_q.start()
            rdma_s.start()

        for c in range(N_CHUNKS):
            rows = pl.ds(c * rows_per, rows_per)
            rdma_q, rdma_s = make_rdmas(c)
            rdma_s.wait()
            rdma_q.wait()
            out_ref[:, rows, :] = (
                q_recv[:, rows, :].astype(jnp.float32) * s_recv[:, rows][:, :, None]
            )

    return pl.pallas_call(
        body,
        out_shape=jax.ShapeDtypeStruct(x.shape, x.dtype),
        in_specs=[
            pl.BlockSpec(memory_space=pltpu.SMEM),
            pl.BlockSpec(memory_space=pltpu.VMEM),
        ],
        out_specs=pl.BlockSpec(memory_space=pltpu.VMEM),
        scratch_shapes=[
            pltpu.VMEM(x.shape, jnp.int8),
            pltpu.VMEM(x.shape, jnp.int8),
            pltpu.VMEM((1, n_rows), jnp.float32),
            pltpu.VMEM((1, n_rows), jnp.float32),
            pltpu.SemaphoreType.DMA((N_CHUNKS,)),
            pltpu.SemaphoreType.DMA((N_CHUNKS,)),
            pltpu.SemaphoreType.DMA((N_CHUNKS,)),
            pltpu.SemaphoreType.DMA((N_CHUNKS,)),
        ],
    )(pi, x)


# device time: 67590 ns/iter; 1.2310x vs baseline; 1.2310x over previous
import jax
import jax.numpy as jnp
from jax import lax
from jax.experimental import pallas as pl
from jax.experimental.pallas import tpu as pltpu

N_CHUNKS = 8


def kernel(x, pi):
    n_rows = x.shape[1]
    rows_per = n_rows // N_CHUNKS

    def body(
        pi_ref,
        x_ref,
        out_ref,
        q_send,
        q_recv,
        s_send,
        s_recv,
        q_send_sems,
        q_recv_sems,
        s_send_sems,
        s_recv_sems,
    ):
        my_pos = lax.axis_index("i")
        dst = pi_ref[my_pos]

        def make_rdmas(c):
            rows = pl.ds(c * rows_per, rows_per)
            rdma_q = pltpu.make_async_remote_copy(
                src_ref=q_send.at[:, rows, :],
                dst_ref=q_recv.at[:, rows, :],
                send_sem=q_send_sems.at[c],
                recv_sem=q_recv_sems.at[c],
                device_id=dst,
                device_id_type=pl.DeviceIdType.LOGICAL,
            )
            rdma_s = pltpu.make_async_remote_copy(
                src_ref=s_send.at[pl.ds(c, 1), :],
                dst_ref=s_recv.at[pl.ds(c, 1), :],
                send_sem=s_send_sems.at[c],
                recv_sem=s_recv_sems.at[c],
                device_id=dst,
                device_id_type=pl.DeviceIdType.LOGICAL,
            )
            return rdma_q, rdma_s

        for c in range(N_CHUNKS):
            rows = pl.ds(c * rows_per, rows_per)
            xv = x_ref[:, rows, :]
            mx = jnp.maximum(jnp.max(jnp.abs(xv)), 1e-30)
            s_send[c, :] = jnp.full((128,), mx / 127.0, jnp.float32)
            q_send[:, rows, :] = jnp.rint(xv * (127.0 / mx)).astype(jnp.int8)
            rdma_q, rdma_s = make_rdmas(c)
            rdma_q.start()
            rdma_s.start()

        for c in range(N_CHUNKS):
            rows = pl.ds(c * rows_per, rows_per)
            rdma_q, rdma_s = make_rdmas(c)
            rdma_s.wait()
            rdma_q.wait()
            out_ref[:, rows, :] = q_recv[:, rows, :].astype(jnp.float32) * s_recv[c, 0]

    return pl.pallas_call(
        body,
        out_shape=jax.ShapeDtypeStruct(x.shape, x.dtype),
        in_specs=[
            pl.BlockSpec(memory_space=pltpu.SMEM),
            pl.BlockSpec(memory_space=pltpu.VMEM),
        ],
        out_specs=pl.BlockSpec(memory_space=pltpu.VMEM),
        scratch_shapes=[
            pltpu.VMEM(x.shape, jnp.int8),
            pltpu.VMEM(x.shape, jnp.int8),
            pltpu.VMEM((N_CHUNKS, 128), jnp.float32),
            pltpu.VMEM((N_CHUNKS, 128), jnp.float32),
            pltpu.SemaphoreType.DMA((N_CHUNKS,)),
            pltpu.SemaphoreType.DMA((N_CHUNKS,)),
            pltpu.SemaphoreType.DMA((N_CHUNKS,)),
            pltpu.SemaphoreType.DMA((N_CHUNKS,)),
        ],
    )(pi, x)


# device time: 58992 ns/iter; 1.4104x vs baseline; 1.1457x over previous
import jax
import jax.numpy as jnp
from jax import lax
from jax.experimental import pallas as pl
from jax.experimental.pallas import tpu as pltpu

N_CHUNKS = 8


def kernel(x, pi):
    n_rows = x.shape[1]
    rows_per = n_rows // N_CHUNKS

    def body(
        pi_ref,
        x_ref,
        out_ref,
        q_send,
        q_recv,
        s_send,
        s_recv,
        q_send_sems,
        q_recv_sems,
        s_send_sems,
        s_recv_sems,
    ):
        my_pos = lax.axis_index("i")
        dst = pi_ref[my_pos]
        src = lax.fori_loop(
            0, 32, lambda k, acc: jnp.where(pi_ref[k] == my_pos, k, acc), 0
        )

        barrier_sem = pltpu.get_barrier_semaphore()
        pl.semaphore_signal(
            barrier_sem, inc=1, device_id=src,
            device_id_type=pl.DeviceIdType.LOGICAL,
        )
        pl.semaphore_wait(barrier_sem, 1)

        def make_rdmas(c):
            rows = pl.ds(c * rows_per, rows_per)
            rdma_q = pltpu.make_async_remote_copy(
                src_ref=q_send.at[:, rows, :],
                dst_ref=q_recv.at[:, rows, :],
                send_sem=q_send_sems.at[c],
                recv_sem=q_recv_sems.at[c],
                device_id=dst,
                device_id_type=pl.DeviceIdType.LOGICAL,
            )
            rdma_s = pltpu.make_async_remote_copy(
                src_ref=s_send.at[pl.ds(c, 1), :],
                dst_ref=s_recv.at[pl.ds(c, 1), :],
                send_sem=s_send_sems.at[c],
                recv_sem=s_recv_sems.at[c],
                device_id=dst,
                device_id_type=pl.DeviceIdType.LOGICAL,
            )
            return rdma_q, rdma_s

        for c in range(N_CHUNKS):
            rows = pl.ds(c * rows_per, rows_per)
            xv = x_ref[:, rows, :]
            mx = jnp.maximum(jnp.max(jnp.abs(xv)), 1e-30)
            s_send[c, :] = jnp.full((128,), mx / 127.0, jnp.float32)
            q_send[:, rows, :] = jnp.rint(xv * (127.0 / mx)).astype(jnp.int8)
            rdma_q, rdma_s = make_rdmas(c)
            rdma_q.start()
            rdma_s.start()

        for c in range(N_CHUNKS):
            rows = pl.ds(c * rows_per, rows_per)
            rdma_q, rdma_s = make_rdmas(c)
            rdma_s.wait()
            rdma_q.wait()
            out_ref[:, rows, :] = q_recv[:, rows, :].astype(jnp.float32) * s_recv[c, 0]

    return pl.pallas_call(
        body,
        out_shape=jax.ShapeDtypeStruct(x.shape, x.dtype),
        in_specs=[
            pl.BlockSpec(memory_space=pltpu.SMEM),
            pl.BlockSpec(memory_space=pltpu.VMEM),
        ],
        out_specs=pl.BlockSpec(memory_space=pltpu.VMEM),
        scratch_shapes=[
            pltpu.VMEM(x.shape, jnp.int8),
            pltpu.VMEM(x.shape, jnp.int8),
            pltpu.VMEM((N_CHUNKS, 128), jnp.float32),
            pltpu.VMEM((N_CHUNKS, 128), jnp.float32),
            pltpu.SemaphoreType.DMA((N_CHUNKS,)),
            pltpu.SemaphoreType.DMA((N_CHUNKS,)),
            pltpu.SemaphoreType.DMA((N_CHUNKS,)),
            pltpu.SemaphoreType.DMA((N_CHUNKS,)),
        ],
        compiler_params=pltpu.CompilerParams(collective_id=0),
    )(pi, x)
